# baseline (speedup 1.0000x reference)
import functools

import jax
import jax.numpy as jnp
from jax import lax
from jax.experimental import pallas as pl

N = 10000
NP = 10240
E = 320000
F_IN = 128
HID = 32
NCLS = 40

_BLK = 1024
_GRID = NP // _BLK


def _dense_body(x, w_ref, asrc_ref, adst_ref, h_ref, as_ref, ad_ref,
                *, heads, cout):
    hfull = jnp.dot(x, w_ref[...], preferred_element_type=jnp.float32)
    asv = []
    adv = []
    for h in range(heads):
        hh = hfull[:, h * cout:(h + 1) * cout]
        if cout == h_ref.shape[2]:
            h_ref[h, :, :] = hh
        else:
            h_ref[h, :, :cout] = hh
            h_ref[h, :, cout:] = jnp.zeros((hh.shape[0], h_ref.shape[2] - cout),
                                           jnp.float32)
        asv.append(jnp.sum(hh * asrc_ref[h][None, :], axis=1))
        adv.append(jnp.sum(hh * adst_ref[h][None, :], axis=1))
    zero = jnp.zeros_like(asv[0])
    while len(asv) < 8:
        asv.append(zero)
        adv.append(zero)
    as_ref[...] = jnp.stack(asv)
    ad_ref[...] = jnp.stack(adv)


def _dense_layer1(x, w, asrc, adst):
    def kfn(x_ref, w_ref, asrc_ref, adst_ref, h_ref, as_ref, ad_ref):
        _dense_body(x_ref[...], w_ref, asrc_ref, adst_ref, h_ref, as_ref,
                    ad_ref, heads=7, cout=HID)

    return pl.pallas_call(
        kfn,
        grid=(_GRID,),
        in_specs=[
            pl.BlockSpec((_BLK, F_IN), lambda i: (i, 0)),
            pl.BlockSpec((F_IN, 7 * HID), lambda i: (0, 0)),
            pl.BlockSpec((7, HID), lambda i: (0, 0)),
            pl.BlockSpec((7, HID), lambda i: (0, 0)),
        ],
        out_specs=[
            pl.BlockSpec((7, _BLK, HID), lambda i: (0, i, 0)),
            pl.BlockSpec((8, _BLK), lambda i: (0, i)),
            pl.BlockSpec((8, _BLK), lambda i: (0, i)),
        ],
        out_shape=[
            jax.ShapeDtypeStruct((7, NP, HID), jnp.float32),
            jax.ShapeDtypeStruct((8, NP), jnp.float32),
            jax.ShapeDtypeStruct((8, NP), jnp.float32),
        ],
    )(x, w, asrc, adst)


def _dense_later(acc_prev, b_prev, w, asrc, adst, *, hp, cp, heads, cout, cpad):
    def kfn(acc_ref, b_ref, w_ref, asrc_ref, adst_ref, h_ref, as_ref, ad_ref):
        parts = [jnp.maximum(acc_ref[h] + b_ref[pl.ds(h * cp, cp)][None, :], 0.0)
                 for h in range(hp)]
        x = jnp.concatenate(parts, axis=1)
        _dense_body(x, w_ref, asrc_ref, adst_ref, h_ref, as_ref, ad_ref,
                    heads=heads, cout=cout)

    return pl.pallas_call(
        kfn,
        grid=(_GRID,),
        in_specs=[
            pl.BlockSpec((hp, _BLK, cp), lambda i: (0, i, 0)),
            pl.BlockSpec((hp * cp,), lambda i: (0,)),
            pl.BlockSpec((hp * cp, heads * cout), lambda i: (0, 0)),
            pl.BlockSpec((heads, cout), lambda i: (0, 0)),
            pl.BlockSpec((heads, cout), lambda i: (0, 0)),
        ],
        out_specs=[
            pl.BlockSpec((heads, _BLK, cpad), lambda i: (0, i, 0)),
            pl.BlockSpec((8, _BLK), lambda i: (0, i)),
            pl.BlockSpec((8, _BLK), lambda i: (0, i)),
        ],
        out_shape=[
            jax.ShapeDtypeStruct((heads, NP, cpad), jnp.float32),
            jax.ShapeDtypeStruct((8, NP), jnp.float32),
            jax.ShapeDtypeStruct((8, NP), jnp.float32),
        ],
    )(acc_prev, b_prev, w, asrc, adst)


def _edge_phase_jnp(h, asn, adn, src, dst, heads, cout):
    al = jax.nn.leaky_relu(asn[:heads, src].T + adn[:heads, dst].T, 0.2)
    amax = jax.ops.segment_max(al, dst, num_segments=N)
    amax = jnp.where(jnp.isfinite(amax), amax, 0.0)
    ex = jnp.exp(al - amax[dst])
    den = jax.ops.segment_sum(ex, dst, num_segments=N)
    coef = ex / (den[dst] + 1e-16)
    hs = h[:, :N, :cout].transpose(1, 0, 2)[src]
    out = jax.ops.segment_sum(hs * coef[:, :, None], dst, num_segments=N)
    out = out.transpose(1, 0, 2)
    pad = jnp.zeros((heads, NP - N, h.shape[2]), jnp.float32)
    outp = out if h.shape[2] == cout else jnp.pad(
        out, ((0, 0), (0, 0), (0, h.shape[2] - cout)))
    return jnp.concatenate([outp, pad], axis=1)


def kernel(x, edge_index, W1, a1_src, a1_dst, b1, W2, a2_src, a2_dst, b2,
           W3, a3_src, a3_dst, b3):
    loop = jnp.arange(N, dtype=edge_index.dtype)
    src = jnp.concatenate([edge_index[0], loop])
    dst = jnp.concatenate([edge_index[1], loop])
    xp = jnp.pad(x, ((0, NP - N), (0, 0)))

    h1, as1, ad1 = _dense_layer1(xp, W1, a1_src, a1_dst)
    acc1 = _edge_phase_jnp(h1, as1, ad1, src, dst, 7, HID)

    h2, as2, ad2 = _dense_later(acc1, b1, W2, a2_src, a2_dst,
                                hp=7, cp=HID, heads=6, cout=HID, cpad=HID)
    acc2 = _edge_phase_jnp(h2, as2, ad2, src, dst, 6, HID)

    h3, as3, ad3 = _dense_later(acc2, b2, W3, a3_src, a3_dst,
                                hp=6, cp=HID, heads=6, cout=NCLS, cpad=48)
    acc3 = _edge_phase_jnp(h3, as3, ad3, src, dst, 6, NCLS)

    return acc3[:, :N, :NCLS].transpose(1, 0, 2).reshape(N, 6 * NCLS) + b3


# speedup vs baseline: 46.5454x; 46.5454x over previous
import functools

import jax
import jax.numpy as jnp
from jax import lax
from jax.experimental import pallas as pl
from jax.experimental.pallas import tpu as pltpu
from jax.experimental.pallas import tpu_sc as plsc

N = 10000
NP = 10240
HALF = 5120
E = 320000
F_IN = 128
HID = 32
NCLS = 40

_BLK = 1024
_GRID = NP // _BLK

ETOT = E + N
ECH = 128
NCHUNK = 162
EPT = NCHUNK * ECH
EPAD = 16 * EPT
ROWS_PT = HALF // 16
DGR = 384
JUNK = 5016


def _dense_body(x, w_ref, asrc_ref, adst_ref, h_ref, as_ref, ad_ref,
                *, heads, cout):
    hfull = jnp.dot(x, w_ref[...], preferred_element_type=jnp.float32)
    asv = []
    adv = []
    for h in range(heads):
        hh = hfull[:, h * cout:(h + 1) * cout]
        if cout == h_ref.shape[2]:
            h_ref[h, :, :] = hh
        else:
            h_ref[h, :, :cout] = hh
            h_ref[h, :, cout:] = jnp.zeros((hh.shape[0], h_ref.shape[2] - cout),
                                           jnp.float32)
        asv.append(jnp.sum(hh * asrc_ref[h][None, :], axis=1))
        adv.append(jnp.sum(hh * adst_ref[h][None, :], axis=1))
    zero = jnp.zeros_like(asv[0])
    while len(asv) < 8:
        asv.append(zero)
        adv.append(zero)
    as_ref[...] = jnp.stack(asv)
    ad_ref[...] = jnp.stack(adv)


def _dense_layer1(x, w, asrc, adst):
    def kfn(x_ref, w_ref, asrc_ref, adst_ref, h_ref, as_ref, ad_ref):
        _dense_body(x_ref[...], w_ref, asrc_ref, adst_ref, h_ref, as_ref,
                    ad_ref, heads=7, cout=HID)

    return pl.pallas_call(
        kfn,
        grid=(_GRID,),
        in_specs=[
            pl.BlockSpec((_BLK, F_IN), lambda i: (i, 0)),
            pl.BlockSpec((F_IN, 7 * HID), lambda i: (0, 0)),
            pl.BlockSpec((7, HID), lambda i: (0, 0)),
            pl.BlockSpec((7, HID), lambda i: (0, 0)),
        ],
        out_specs=[
            pl.BlockSpec((7, _BLK, HID), lambda i: (0, i, 0)),
            pl.BlockSpec((8, _BLK), lambda i: (0, i)),
            pl.BlockSpec((8, _BLK), lambda i: (0, i)),
        ],
        out_shape=[
            jax.ShapeDtypeStruct((7, NP, HID), jnp.float32),
            jax.ShapeDtypeStruct((8, NP), jnp.float32),
            jax.ShapeDtypeStruct((8, NP), jnp.float32),
        ],
    )(x, w, asrc, adst)


def _dense_later(acc_prev, b_prev, w, asrc, adst, *, hp, cp, heads, cout, cpad):
    def kfn(acc_ref, b_ref, w_ref, asrc_ref, adst_ref, h_ref, as_ref, ad_ref):
        parts = [jnp.maximum(acc_ref[h] + b_ref[pl.ds(h * cp, cp)][None, :], 0.0)
                 for h in range(hp)]
        x = jnp.concatenate(parts, axis=1)
        _dense_body(x, w_ref, asrc_ref, adst_ref, h_ref, as_ref, ad_ref,
                    heads=heads, cout=cout)

    return pl.pallas_call(
        kfn,
        grid=(_GRID,),
        in_specs=[
            pl.BlockSpec((hp, _BLK, cp), lambda i: (0, i, 0)),
            pl.BlockSpec((hp * cp,), lambda i: (0,)),
            pl.BlockSpec((hp * cp, heads * cout), lambda i: (0, 0)),
            pl.BlockSpec((heads, cout), lambda i: (0, 0)),
            pl.BlockSpec((heads, cout), lambda i: (0, 0)),
        ],
        out_specs=[
            pl.BlockSpec((heads, _BLK, cpad), lambda i: (0, i, 0)),
            pl.BlockSpec((8, _BLK), lambda i: (0, i)),
            pl.BlockSpec((8, _BLK), lambda i: (0, i)),
        ],
        out_shape=[
            jax.ShapeDtypeStruct((heads, NP, cpad), jnp.float32),
            jax.ShapeDtypeStruct((8, NP), jnp.float32),
            jax.ShapeDtypeStruct((8, NP), jnp.float32),
        ],
    )(acc_prev, b_prev, w, asrc, adst)


def _sc_edge_layer(h, asn, adn, srcr, dstr, heads, cpad):
    mesh = plsc.VectorSubcoreMesh(core_axis_name="c", subcore_axis_name="s")

    @functools.partial(
        pl.kernel,
        out_type=jax.ShapeDtypeStruct((heads, 2, HALF, cpad), jnp.float32),
        mesh=mesh,
        compiler_params=pltpu.CompilerParams(needs_layout_passes=False,
                                             use_tc_tiling_on_sc=False),
        scratch_types=[
            pltpu.VMEM((NCHUNK, ECH), jnp.int32),
            pltpu.VMEM((NCHUNK, ECH), jnp.int32),
            pltpu.VMEM((NCHUNK, ECH), jnp.float32),
            pltpu.VMEM((NP,), jnp.float32),
            pltpu.VMEM((NP,), jnp.float32),
            pltpu.VMEM((DGR, 16), jnp.float32),
            pltpu.VMEM((ECH, cpad), jnp.float32),
            pltpu.VMEM((ECH, cpad), jnp.float32),
            pltpu.VMEM((ECH,), jnp.float32),
            pltpu.VMEM((DGR // ECH, ECH), jnp.int32),
            pltpu.VMEM((DGR // 16, 16), jnp.float32),
            pltpu.VMEM_SHARED((HALF, cpad), jnp.float32),
            pltpu.VMEM_SHARED((DGR, 16), jnp.float32),
        ],
    )
    def k(h_hbm, as_hbm, ad_hbm, src_hbm, dst_hbm, out_hbm,
          src_v, dst_v, ex_v, as_v, ad_v, den_v, rows_v, zrow_v, coef_v,
          iota_v, zq_v, acc_sh, den_sh):
        c = lax.axis_index("c")
        s = lax.axis_index("s")
        zv = jnp.zeros((16,), jnp.float32)
        pltpu.sync_copy(src_hbm.at[s], src_v)
        pltpu.sync_copy(dst_hbm.at[s], dst_v)

        def perm_body(ci, _):
            for jj in range(ECH // 16):
                sl = pl.ds(jj * 16, 16)
                sv = src_v[ci, sl]
                dv = dst_v[ci, sl]
                ps = (jnp.bitwise_and(sv, 1) * HALF
                      + jnp.right_shift(sv, 1))
                dl = jnp.where(jnp.bitwise_and(dv, 1) == c,
                               jnp.right_shift(dv, 1),
                               jnp.full((16,), JUNK, jnp.int32))
                src_v[ci, sl] = ps
                dst_v[ci, sl] = dl
            return 0
        lax.fori_loop(0, NCHUNK, perm_body, 0)

        def zrow_body(j, _):
            for w in range(cpad // 16):
                zrow_v[j, pl.ds(w * 16, 16)] = zv
            return 0
        lax.fori_loop(0, ECH, zrow_body, 0)

        def zq_body(j, _):
            zq_v[j, :] = zv
            return 0
        lax.fori_loop(0, DGR // 16, zq_body, 0)

        for q in range(DGR // ECH):
            for jj in range(ECH // 16):
                iota_v[q, pl.ds(jj * 16, 16)] = (
                    lax.iota(jnp.int32, 16) + (q * ECH + jj * 16))

        def head_body(head, _):
            base = s * ROWS_PT
            pltpu.sync_copy(as_hbm.at[head], as_v)
            pltpu.sync_copy(ad_hbm.at[head], ad_v)
            lane8 = lax.iota(jnp.int32, 16) < 8
            for zb in (4992, 10112):
                for ref in (as_v, ad_v):
                    v = ref[pl.ds(zb, 16)]
                    ref[pl.ds(zb, 16)] = jnp.where(lane8, v, 0.0)
                    for t in range(zb + 16, zb + 128, 16):
                        ref[pl.ds(t, 16)] = zv
            pltpu.sync_copy(zrow_v, acc_sh.at[pl.ds(base, ECH)])
            pltpu.sync_copy(zrow_v, acc_sh.at[pl.ds(base + ECH, ECH)])
            pltpu.sync_copy(zrow_v.at[pl.ds(0, 64)],
                            acc_sh.at[pl.ds(base + 2 * ECH, 64)])
            pltpu.sync_copy(zq_v,
                            den_sh.at[pl.ds(s * (DGR // 16), DGR // 16)])

            def zden(t, _):
                den_v[t, :] = zv
                return 0
            lax.fori_loop(0, DGR, zden, 0)

            def mbody(t, carry):
                ma, md = carry
                ma = jnp.maximum(ma, as_v[pl.ds(t * 16, 16)])
                md = jnp.maximum(md, ad_v[pl.ds(t * 16, 16)])
                return ma, md
            minit = jnp.full((16,), -1e30, jnp.float32)
            ma, md = lax.fori_loop(0, NP // 16, mbody, (minit, minit))
            msa = ma[0]
            msd = md[0]
            for l in range(1, 16):
                msa = jnp.maximum(msa, ma[l])
                msd = jnp.maximum(msd, md[l])
            mm = msa + msd
            m = jnp.where(mm > 0.0, mm, 0.2 * mm)

            plsc.subcore_barrier()

            coff = c * HALF

            def p1(ci, _):
                for jj in range(ECH // 16):
                    sl = pl.ds(jj * 16, 16)
                    ps = src_v[ci, sl]
                    dl = dst_v[ci, sl]
                    pd = dl + coff
                    a = (plsc.load_gather(as_v, [ps])
                         + plsc.load_gather(ad_v, [pd]))
                    a = jnp.where(a > 0.0, a, 0.2 * a)
                    ev = jnp.exp(a - m)
                    ex_v[ci, sl] = ev
                    plsc.addupdate_scatter(
                        den_v, [jnp.right_shift(dl, 4),
                                jnp.bitwise_and(dl, 15)], ev)
                return 0
            lax.fori_loop(0, NCHUNK, p1, 0)

            for q in range(DGR // ECH):
                pltpu.sync_copy(den_v.at[pl.ds(q * ECH, ECH)],
                                den_sh.at[iota_v.at[q]], add=True)
            plsc.subcore_barrier()
            pltpu.sync_copy(den_sh, den_v)

            def p2(ci, _):
                pltpu.sync_copy(h_hbm.at[head].at[src_v.at[ci]], rows_v)
                for jj in range(ECH // 16):
                    sl = pl.ds(jj * 16, 16)
                    dl = dst_v[ci, sl]
                    dg = plsc.load_gather(
                        den_v, [jnp.right_shift(dl, 4),
                                jnp.bitwise_and(dl, 15)])
                    coef_v[sl] = ex_v[ci, sl] / (dg + 1e-16)
                for jj in range(ECH // 16):
                    cvec = coef_v[pl.ds(jj * 16, 16)]
                    for l in range(16):
                        j = jj * 16 + l
                        cf = cvec[l]
                        for w in range(cpad // 16):
                            sl2 = pl.ds(w * 16, 16)
                            rows_v[j, sl2] = rows_v[j, sl2] * cf
                pltpu.sync_copy(rows_v, acc_sh.at[dst_v.at[ci]], add=True)
                return 0
            lax.fori_loop(0, NCHUNK, p2, 0)
            plsc.subcore_barrier()

            pltpu.sync_copy(acc_sh.at[pl.ds(base, ROWS_PT)],
                            out_hbm.at[head, c, pl.ds(base, ROWS_PT)])
            plsc.subcore_barrier()
            return 0

        lax.fori_loop(0, heads, head_body, 0)

    return k(h, asn, adn, srcr, dstr)


def kernel(x, edge_index, W1, a1_src, a1_dst, b1, W2, a2_src, a2_dst, b2,
           W3, a3_src, a3_dst, b3):
    loop = jnp.arange(N, dtype=edge_index.dtype)
    sent = jnp.full((EPAD - ETOT,), N, dtype=edge_index.dtype)
    srcr = jnp.concatenate([edge_index[0], loop, sent]).reshape(16, NCHUNK, ECH)
    dstr = jnp.concatenate([edge_index[1], loop, sent]).reshape(16, NCHUNK, ECH)

    z = jnp.zeros((HALF - 5000, F_IN), jnp.float32)
    xp = jnp.concatenate([x[0::2], z, x[1::2], z])

    h1, as1, ad1 = _dense_layer1(xp, W1, a1_src, a1_dst)
    acc1 = _sc_edge_layer(h1, as1, ad1, srcr, dstr, 7, HID)
    acc1 = acc1.reshape(7, NP, HID)

    h2, as2, ad2 = _dense_later(acc1, b1, W2, a2_src, a2_dst,
                                hp=7, cp=HID, heads=6, cout=HID, cpad=HID)
    acc2 = _sc_edge_layer(h2, as2, ad2, srcr, dstr, 6, HID).reshape(6, NP, HID)

    h3, as3, ad3 = _dense_later(acc2, b2, W3, a3_src, a3_dst,
                                hp=6, cp=HID, heads=6, cout=NCLS, cpad=48)
    acc3 = _sc_edge_layer(h3, as3, ad3, srcr, dstr, 6, 48)

    out = acc3[:, :, :5000, :NCLS]
    out = out.transpose(2, 1, 0, 3)
    return out.reshape(N, 6 * NCLS) + b3


# speedup vs baseline: 49.5021x; 1.0635x over previous
import functools

import jax
import jax.numpy as jnp
from jax import lax
from jax.experimental import pallas as pl
from jax.experimental.pallas import tpu as pltpu
from jax.experimental.pallas import tpu_sc as plsc

N = 10000
NP = 10240
HALF = 5120
E = 320000
F_IN = 128
HID = 32
NCLS = 40

_BLK = 1024
_GRID = NP // _BLK

ETOT = E + N
ECH = 128
NCHUNK = 162
EPT = NCHUNK * ECH
EPAD = 16 * EPT
ROWS_PT = HALF // 16
DGR = 384
JUNK = 5016


def _dense_body(x, w_ref, asrc_ref, adst_ref, h_ref, as_ref, ad_ref,
                *, heads, cout):
    hfull = jnp.dot(x, w_ref[...], preferred_element_type=jnp.float32)
    asv = []
    adv = []
    for h in range(heads):
        hh = hfull[:, h * cout:(h + 1) * cout]
        if cout == h_ref.shape[2]:
            h_ref[h, :, :] = hh
        else:
            h_ref[h, :, :cout] = hh
            h_ref[h, :, cout:] = jnp.zeros((hh.shape[0], h_ref.shape[2] - cout),
                                           jnp.float32)
        asv.append(jnp.sum(hh * asrc_ref[h][None, :], axis=1))
        adv.append(jnp.sum(hh * adst_ref[h][None, :], axis=1))
    zero = jnp.zeros_like(asv[0])
    while len(asv) < 8:
        asv.append(zero)
        adv.append(zero)
    as_ref[...] = jnp.stack(asv)
    ad_ref[...] = jnp.stack(adv)


def _dense_layer1(x, w, asrc, adst):
    def kfn(x_ref, w_ref, asrc_ref, adst_ref, h_ref, as_ref, ad_ref):
        _dense_body(x_ref[...], w_ref, asrc_ref, adst_ref, h_ref, as_ref,
                    ad_ref, heads=7, cout=HID)

    return pl.pallas_call(
        kfn,
        grid=(_GRID,),
        in_specs=[
            pl.BlockSpec((_BLK, F_IN), lambda i: (i, 0)),
            pl.BlockSpec((F_IN, 7 * HID), lambda i: (0, 0)),
            pl.BlockSpec((7, HID), lambda i: (0, 0)),
            pl.BlockSpec((7, HID), lambda i: (0, 0)),
        ],
        out_specs=[
            pl.BlockSpec((7, _BLK, HID), lambda i: (0, i, 0)),
            pl.BlockSpec((8, _BLK), lambda i: (0, i)),
            pl.BlockSpec((8, _BLK), lambda i: (0, i)),
        ],
        out_shape=[
            jax.ShapeDtypeStruct((7, NP, HID), jnp.float32),
            jax.ShapeDtypeStruct((8, NP), jnp.float32),
            jax.ShapeDtypeStruct((8, NP), jnp.float32),
        ],
    )(x, w, asrc, adst)


def _dense_later(acc_prev, b_prev, w, asrc, adst, *, hp, cp, heads, cout, cpad):
    def kfn(acc_ref, b_ref, w_ref, asrc_ref, adst_ref, h_ref, as_ref, ad_ref):
        parts = [jnp.maximum(acc_ref[h] + b_ref[pl.ds(h * cp, cp)][None, :], 0.0)
                 for h in range(hp)]
        x = jnp.concatenate(parts, axis=1)
        _dense_body(x, w_ref, asrc_ref, adst_ref, h_ref, as_ref, ad_ref,
                    heads=heads, cout=cout)

    return pl.pallas_call(
        kfn,
        grid=(_GRID,),
        in_specs=[
            pl.BlockSpec((hp, _BLK, cp), lambda i: (0, i, 0)),
            pl.BlockSpec((hp * cp,), lambda i: (0,)),
            pl.BlockSpec((hp * cp, heads * cout), lambda i: (0, 0)),
            pl.BlockSpec((heads, cout), lambda i: (0, 0)),
            pl.BlockSpec((heads, cout), lambda i: (0, 0)),
        ],
        out_specs=[
            pl.BlockSpec((heads, _BLK, cpad), lambda i: (0, i, 0)),
            pl.BlockSpec((8, _BLK), lambda i: (0, i)),
            pl.BlockSpec((8, _BLK), lambda i: (0, i)),
        ],
        out_shape=[
            jax.ShapeDtypeStruct((heads, NP, cpad), jnp.float32),
            jax.ShapeDtypeStruct((8, NP), jnp.float32),
            jax.ShapeDtypeStruct((8, NP), jnp.float32),
        ],
    )(acc_prev, b_prev, w, asrc, adst)


def _sc_edge_layer(h, asn, adn, srcr, dstr, heads, cpad):
    mesh = plsc.VectorSubcoreMesh(core_axis_name="c", subcore_axis_name="s")

    @functools.partial(
        pl.kernel,
        out_type=jax.ShapeDtypeStruct((heads, 2, HALF, cpad), jnp.float32),
        mesh=mesh,
        compiler_params=pltpu.CompilerParams(needs_layout_passes=False,
                                             use_tc_tiling_on_sc=False),
        scratch_types=[
            pltpu.VMEM((NCHUNK, ECH), jnp.int32),
            pltpu.VMEM((NCHUNK, ECH), jnp.int32),
            pltpu.VMEM((NCHUNK, ECH), jnp.float32),
            pltpu.VMEM((NP,), jnp.float32),
            pltpu.VMEM((NP,), jnp.float32),
            pltpu.VMEM((DGR, 16), jnp.float32),
            pltpu.VMEM((ECH, cpad), jnp.float32),
            pltpu.VMEM((ECH, cpad), jnp.float32),
            pltpu.VMEM((ECH, cpad), jnp.float32),
            pltpu.VMEM((ECH,), jnp.float32),
            pltpu.SemaphoreType.DMA,
            pltpu.SemaphoreType.DMA,
            pltpu.VMEM((DGR // ECH, ECH), jnp.int32),
            pltpu.VMEM((DGR // 16, 16), jnp.float32),
            pltpu.VMEM_SHARED((HALF, cpad), jnp.float32),
            pltpu.VMEM_SHARED((DGR, 16), jnp.float32),
        ],
    )
    def k(h_hbm, as_hbm, ad_hbm, src_hbm, dst_hbm, out_hbm,
          src_v, dst_v, ex_v, as_v, ad_v, den_v, rows_v, rows_w, zrow_v,
          coef_v, sem_a, sem_b, iota_v, zq_v, acc_sh, den_sh):
        c = lax.axis_index("c")
        s = lax.axis_index("s")
        zv = jnp.zeros((16,), jnp.float32)
        pltpu.sync_copy(src_hbm.at[s], src_v)
        pltpu.sync_copy(dst_hbm.at[s], dst_v)

        def perm_body(ci, _):
            for jj in range(ECH // 16):
                sl = pl.ds(jj * 16, 16)
                sv = src_v[ci, sl]
                dv = dst_v[ci, sl]
                ps = (jnp.bitwise_and(sv, 1) * HALF
                      + jnp.right_shift(sv, 1))
                dl = jnp.where(jnp.bitwise_and(dv, 1) == c,
                               jnp.right_shift(dv, 1),
                               jnp.full((16,), JUNK, jnp.int32))
                src_v[ci, sl] = ps
                dst_v[ci, sl] = dl
            return 0
        lax.fori_loop(0, NCHUNK, perm_body, 0)

        def zrow_body(j, _):
            for w in range(cpad // 16):
                zrow_v[j, pl.ds(w * 16, 16)] = zv
            return 0
        lax.fori_loop(0, ECH, zrow_body, 0)

        def zq_body(j, _):
            zq_v[j, :] = zv
            return 0
        lax.fori_loop(0, DGR // 16, zq_body, 0)

        for q in range(DGR // ECH):
            for jj in range(ECH // 16):
                iota_v[q, pl.ds(jj * 16, 16)] = (
                    lax.iota(jnp.int32, 16) + (q * ECH + jj * 16))

        def head_body(head, _):
            base = s * ROWS_PT
            pltpu.sync_copy(as_hbm.at[head], as_v)
            pltpu.sync_copy(ad_hbm.at[head], ad_v)
            lane8 = lax.iota(jnp.int32, 16) < 8
            for zb in (4992, 10112):
                for ref in (as_v, ad_v):
                    v = ref[pl.ds(zb, 16)]
                    ref[pl.ds(zb, 16)] = jnp.where(lane8, v, 0.0)
                    for t in range(zb + 16, zb + 128, 16):
                        ref[pl.ds(t, 16)] = zv
            pltpu.sync_copy(zrow_v, acc_sh.at[pl.ds(base, ECH)])
            pltpu.sync_copy(zrow_v, acc_sh.at[pl.ds(base + ECH, ECH)])
            pltpu.sync_copy(zrow_v.at[pl.ds(0, 64)],
                            acc_sh.at[pl.ds(base + 2 * ECH, 64)])
            pltpu.sync_copy(zq_v,
                            den_sh.at[pl.ds(s * (DGR // 16), DGR // 16)])

            def zden(t, _):
                den_v[t, :] = zv
                return 0
            lax.fori_loop(0, DGR, zden, 0)

            def mbody(t, carry):
                ma, md = carry
                ma = jnp.maximum(ma, as_v[pl.ds(t * 16, 16)])
                md = jnp.maximum(md, ad_v[pl.ds(t * 16, 16)])
                return ma, md
            minit = jnp.full((16,), -1e30, jnp.float32)
            ma, md = lax.fori_loop(0, NP // 16, mbody, (minit, minit))
            msa = ma[0]
            msd = md[0]
            for l in range(1, 16):
                msa = jnp.maximum(msa, ma[l])
                msd = jnp.maximum(msd, md[l])
            mm = msa + msd
            m = jnp.where(mm > 0.0, mm, 0.2 * mm)

            plsc.subcore_barrier()

            coff = c * HALF

            def p1(ci, _):
                for jj in range(ECH // 16):
                    sl = pl.ds(jj * 16, 16)
                    ps = src_v[ci, sl]
                    dl = dst_v[ci, sl]
                    pd = dl + coff
                    a = (plsc.load_gather(as_v, [ps])
                         + plsc.load_gather(ad_v, [pd]))
                    a = jnp.where(a > 0.0, a, 0.2 * a)
                    ev = jnp.exp(a - m)
                    ex_v[ci, sl] = ev
                    plsc.addupdate_scatter(
                        den_v, [jnp.right_shift(dl, 4),
                                jnp.bitwise_and(dl, 15)], ev)
                return 0
            lax.fori_loop(0, NCHUNK, p1, 0)

            for q in range(DGR // ECH):
                pltpu.sync_copy(den_v.at[pl.ds(q * ECH, ECH)],
                                den_sh.at[iota_v.at[q]], add=True)
            plsc.subcore_barrier()
            pltpu.sync_copy(den_sh, den_v)

            def gstart(ci, buf, sem):
                pltpu.async_copy(h_hbm.at[head].at[src_v.at[ci]], buf, sem)

            def gwait(buf, sem):
                pltpu.make_async_copy(h_hbm.at[head].at[src_v.at[0]],
                                      buf, sem).wait()

            def chunk_compute(ci, buf, sem):
                for jj in range(ECH // 16):
                    sl = pl.ds(jj * 16, 16)
                    dl = dst_v[ci, sl]
                    dg = plsc.load_gather(
                        den_v, [jnp.right_shift(dl, 4),
                                jnp.bitwise_and(dl, 15)])
                    coef_v[sl] = ex_v[ci, sl] / (dg + 1e-16)
                gwait(buf, sem)
                for jj in range(ECH // 16):
                    cvec = coef_v[pl.ds(jj * 16, 16)]
                    for l in range(16):
                        j = jj * 16 + l
                        cf = cvec[l]
                        for w in range(cpad // 16):
                            sl2 = pl.ds(w * 16, 16)
                            buf[j, sl2] = buf[j, sl2] * cf
                pltpu.sync_copy(buf, acc_sh.at[dst_v.at[ci]], add=True)

            gstart(0, rows_v, sem_a)
            gstart(1, rows_w, sem_b)

            def p2pair(cj, _):
                ci0 = 2 * cj
                chunk_compute(ci0, rows_v, sem_a)

                @pl.when(ci0 + 2 < NCHUNK)
                def _():
                    gstart(ci0 + 2, rows_v, sem_a)
                chunk_compute(ci0 + 1, rows_w, sem_b)

                @pl.when(ci0 + 3 < NCHUNK)
                def _():
                    gstart(ci0 + 3, rows_w, sem_b)
                return 0
            lax.fori_loop(0, NCHUNK // 2, p2pair, 0)
            plsc.subcore_barrier()

            pltpu.sync_copy(acc_sh.at[pl.ds(base, ROWS_PT)],
                            out_hbm.at[head, c, pl.ds(base, ROWS_PT)])
            plsc.subcore_barrier()
            return 0

        lax.fori_loop(0, heads, head_body, 0)

    return k(h, asn, adn, srcr, dstr)


def kernel(x, edge_index, W1, a1_src, a1_dst, b1, W2, a2_src, a2_dst, b2,
           W3, a3_src, a3_dst, b3):
    loop = jnp.arange(N, dtype=edge_index.dtype)
    sent = jnp.full((EPAD - ETOT,), N, dtype=edge_index.dtype)
    srcr = jnp.concatenate([edge_index[0], loop, sent]).reshape(16, NCHUNK, ECH)
    dstr = jnp.concatenate([edge_index[1], loop, sent]).reshape(16, NCHUNK, ECH)

    z = jnp.zeros((HALF - 5000, F_IN), jnp.float32)
    xp = jnp.concatenate([x[0::2], z, x[1::2], z])

    h1, as1, ad1 = _dense_layer1(xp, W1, a1_src, a1_dst)
    acc1 = _sc_edge_layer(h1, as1, ad1, srcr, dstr, 7, HID)
    acc1 = acc1.reshape(7, NP, HID)

    h2, as2, ad2 = _dense_later(acc1, b1, W2, a2_src, a2_dst,
                                hp=7, cp=HID, heads=6, cout=HID, cpad=HID)
    acc2 = _sc_edge_layer(h2, as2, ad2, srcr, dstr, 6, HID).reshape(6, NP, HID)

    h3, as3, ad3 = _dense_later(acc2, b2, W3, a3_src, a3_dst,
                                hp=6, cp=HID, heads=6, cout=NCLS, cpad=48)
    acc3 = _sc_edge_layer(h3, as3, ad3, srcr, dstr, 6, 48)

    out = acc3[:, :, :5000, :NCLS]
    out = out.transpose(2, 1, 0, 3)
    return out.reshape(N, 6 * NCLS) + b3


# speedup vs baseline: 102.9082x; 2.0789x over previous
import functools

import jax
import jax.numpy as jnp
from jax import lax
from jax.experimental import pallas as pl
from jax.experimental.pallas import tpu as pltpu
from jax.experimental.pallas import tpu_sc as plsc

N = 10000
NP = 10240
HALF = 5120
E = 320000
F_IN = 128
HID = 32
NCLS = 40

_BLK = 1024
_GRID = NP // _BLK

ETOT = E + N
ECH = 128
NCHUNK = 162
EPT = NCHUNK * ECH
EPAD = 16 * EPT
ROWS_PT = HALF // 16
DGR = 384
JUNK = 5016


def _dense_body(x, w_ref, asrc_ref, adst_ref, h_ref, as_ref, ad_ref,
                *, heads, cout):
    hfull = jnp.dot(x, w_ref[...], preferred_element_type=jnp.float32)
    asv = []
    adv = []
    for h in range(heads):
        hh = hfull[:, h * cout:(h + 1) * cout]
        if cout == h_ref.shape[2]:
            h_ref[h, :, :] = hh
        else:
            h_ref[h, :, :cout] = hh
            h_ref[h, :, cout:] = jnp.zeros((hh.shape[0], h_ref.shape[2] - cout),
                                           jnp.float32)
        asv.append(jnp.sum(hh * asrc_ref[h][None, :], axis=1))
        adv.append(jnp.sum(hh * adst_ref[h][None, :], axis=1))
    zero = jnp.zeros_like(asv[0])
    while len(asv) < 8:
        asv.append(zero)
        adv.append(zero)
    as_ref[...] = jnp.stack(asv)
    ad_ref[...] = jnp.stack(adv)


def _dense_layer1(x, w, asrc, adst):
    def kfn(x_ref, w_ref, asrc_ref, adst_ref, h_ref, as_ref, ad_ref):
        _dense_body(x_ref[...], w_ref, asrc_ref, adst_ref, h_ref, as_ref,
                    ad_ref, heads=7, cout=HID)

    return pl.pallas_call(
        kfn,
        grid=(_GRID,),
        in_specs=[
            pl.BlockSpec((_BLK, F_IN), lambda i: (i, 0)),
            pl.BlockSpec((F_IN, 7 * HID), lambda i: (0, 0)),
            pl.BlockSpec((7, HID), lambda i: (0, 0)),
            pl.BlockSpec((7, HID), lambda i: (0, 0)),
        ],
        out_specs=[
            pl.BlockSpec((7, _BLK, HID), lambda i: (0, i, 0)),
            pl.BlockSpec((8, _BLK), lambda i: (0, i)),
            pl.BlockSpec((8, _BLK), lambda i: (0, i)),
        ],
        out_shape=[
            jax.ShapeDtypeStruct((7, NP, HID), jnp.float32),
            jax.ShapeDtypeStruct((8, NP), jnp.float32),
            jax.ShapeDtypeStruct((8, NP), jnp.float32),
        ],
    )(x, w, asrc, adst)


def _dense_later(acc_prev, b_prev, w, asrc, adst, *, hp, cp, heads, cout, cpad):
    def kfn(acc_ref, b_ref, w_ref, asrc_ref, adst_ref, h_ref, as_ref, ad_ref):
        parts = [jnp.maximum(acc_ref[h] + b_ref[pl.ds(h * cp, cp)][None, :], 0.0)
                 for h in range(hp)]
        x = jnp.concatenate(parts, axis=1)
        _dense_body(x, w_ref, asrc_ref, adst_ref, h_ref, as_ref, ad_ref,
                    heads=heads, cout=cout)

    return pl.pallas_call(
        kfn,
        grid=(_GRID,),
        in_specs=[
            pl.BlockSpec((hp, _BLK, cp), lambda i: (0, i, 0)),
            pl.BlockSpec((hp * cp,), lambda i: (0,)),
            pl.BlockSpec((hp * cp, heads * cout), lambda i: (0, 0)),
            pl.BlockSpec((heads, cout), lambda i: (0, 0)),
            pl.BlockSpec((heads, cout), lambda i: (0, 0)),
        ],
        out_specs=[
            pl.BlockSpec((heads, _BLK, cpad), lambda i: (0, i, 0)),
            pl.BlockSpec((8, _BLK), lambda i: (0, i)),
            pl.BlockSpec((8, _BLK), lambda i: (0, i)),
        ],
        out_shape=[
            jax.ShapeDtypeStruct((heads, NP, cpad), jnp.float32),
            jax.ShapeDtypeStruct((8, NP), jnp.float32),
            jax.ShapeDtypeStruct((8, NP), jnp.float32),
        ],
    )(acc_prev, b_prev, w, asrc, adst)


def _sc_edge_layer(h, asn, adn, srcr, dstr, heads, cpad):
    mesh = plsc.VectorSubcoreMesh(core_axis_name="c", subcore_axis_name="s")

    @functools.partial(
        pl.kernel,
        out_type=jax.ShapeDtypeStruct((heads, 2, HALF, cpad), jnp.float32),
        mesh=mesh,
        compiler_params=pltpu.CompilerParams(needs_layout_passes=False,
                                             use_tc_tiling_on_sc=False),
        scratch_types=[
            pltpu.VMEM((NCHUNK, ECH), jnp.int32),
            pltpu.VMEM((NCHUNK, ECH), jnp.int32),
            pltpu.VMEM((NCHUNK, ECH), jnp.float32),
            pltpu.VMEM((NP,), jnp.float32),
            pltpu.VMEM((NP,), jnp.float32),
            pltpu.VMEM((DGR, 16), jnp.float32),
            pltpu.VMEM((ECH, cpad), jnp.float32),
            pltpu.VMEM((ECH, cpad), jnp.float32),
            pltpu.VMEM((ECH, cpad), jnp.float32),
            pltpu.VMEM((ECH,), jnp.float32),
            pltpu.SemaphoreType.DMA,
            pltpu.SemaphoreType.DMA,
            pltpu.VMEM((DGR // ECH, ECH), jnp.int32),
            pltpu.VMEM((DGR // 16, 16), jnp.float32),
            pltpu.VMEM_SHARED((HALF, cpad), jnp.float32),
            pltpu.VMEM_SHARED((DGR, 16), jnp.float32),
        ],
    )
    def k(h_hbm, as_hbm, ad_hbm, src_hbm, dst_hbm, out_hbm,
          src_v, dst_v, ex_v, as_v, ad_v, den_v, rows_v, rows_w, zrow_v,
          coef_v, sem_a, sem_b, iota_v, zq_v, acc_sh, den_sh):
        c = lax.axis_index("c")
        s = lax.axis_index("s")
        zv = jnp.zeros((16,), jnp.float32)
        pltpu.sync_copy(src_hbm.at[s], src_v)
        pltpu.sync_copy(dst_hbm.at[s], dst_v)

        iota16 = lax.iota(jnp.int32, 16)

        def perm_body(v, wp):
            ci = jnp.right_shift(v, 3)
            jo = jnp.bitwise_and(v, 7) * 16
            sv = src_v[ci, pl.ds(jo, 16)]
            dv = dst_v[ci, pl.ds(jo, 16)]
            keep = jnp.bitwise_and(dv, 1) == c
            ps = jnp.bitwise_and(sv, 1) * HALF + jnp.right_shift(sv, 1)
            dl = jnp.right_shift(dv, 1)
            pos = wp - 1 + plsc.cumsum(keep.astype(jnp.int32))
            plsc.store_scatter(src_v, [jnp.right_shift(pos, 7),
                                       jnp.bitwise_and(pos, 127)], ps,
                               mask=keep)
            plsc.store_scatter(dst_v, [jnp.right_shift(pos, 7),
                                       jnp.bitwise_and(pos, 127)], dl,
                               mask=keep)
            cnt = plsc.all_reduce_population_count(keep)
            return wp + cnt[0]
        la = lax.fori_loop(0, NCHUNK * (ECH // 16), perm_body,
                           jnp.int32(0))
        nca = jnp.right_shift(la + 127, 7)
        for kpad in range(8):
            pidx = la + kpad * 16 + iota16
            pmsk = pidx < nca * ECH
            plsc.store_scatter(src_v, [jnp.right_shift(pidx, 7),
                                       jnp.bitwise_and(pidx, 127)],
                               jnp.full((16,), 5000, jnp.int32) + c * HALF,
                               mask=pmsk)
            plsc.store_scatter(dst_v, [jnp.right_shift(pidx, 7),
                                       jnp.bitwise_and(pidx, 127)],
                               jnp.full((16,), JUNK, jnp.int32),
                               mask=pmsk)

        def zrow_body(j, _):
            for w in range(cpad // 16):
                zrow_v[j, pl.ds(w * 16, 16)] = zv
            return 0
        lax.fori_loop(0, ECH, zrow_body, 0)

        def zq_body(j, _):
            zq_v[j, :] = zv
            return 0
        lax.fori_loop(0, DGR // 16, zq_body, 0)

        for q in range(DGR // ECH):
            for jj in range(ECH // 16):
                iota_v[q, pl.ds(jj * 16, 16)] = (
                    lax.iota(jnp.int32, 16) + (q * ECH + jj * 16))

        def head_body(head, _):
            base = s * ROWS_PT
            pltpu.sync_copy(as_hbm.at[head], as_v)
            pltpu.sync_copy(ad_hbm.at[head], ad_v)
            lane8 = lax.iota(jnp.int32, 16) < 8
            for zb in (4992, 10112):
                for ref in (as_v, ad_v):
                    v = ref[pl.ds(zb, 16)]
                    ref[pl.ds(zb, 16)] = jnp.where(lane8, v, 0.0)
                    for t in range(zb + 16, zb + 128, 16):
                        ref[pl.ds(t, 16)] = zv
            pltpu.sync_copy(zrow_v, acc_sh.at[pl.ds(base, ECH)])
            pltpu.sync_copy(zrow_v, acc_sh.at[pl.ds(base + ECH, ECH)])
            pltpu.sync_copy(zrow_v.at[pl.ds(0, 64)],
                            acc_sh.at[pl.ds(base + 2 * ECH, 64)])
            pltpu.sync_copy(zq_v,
                            den_sh.at[pl.ds(s * (DGR // 16), DGR // 16)])

            def zden(t, _):
                den_v[t, :] = zv
                return 0
            lax.fori_loop(0, DGR, zden, 0)

            def mbody(t, carry):
                ma, md = carry
                ma = jnp.maximum(ma, as_v[pl.ds(t * 16, 16)])
                md = jnp.maximum(md, ad_v[pl.ds(t * 16, 16)])
                return ma, md
            minit = jnp.full((16,), -1e30, jnp.float32)
            ma, md = lax.fori_loop(0, NP // 16, mbody, (minit, minit))
            msa = ma[0]
            msd = md[0]
            for l in range(1, 16):
                msa = jnp.maximum(msa, ma[l])
                msd = jnp.maximum(msd, md[l])
            mm = msa + msd
            m = jnp.where(mm > 0.0, mm, 0.2 * mm)

            plsc.subcore_barrier()

            coff = c * HALF

            def p1(ci, _):
                for jj in range(ECH // 16):
                    sl = pl.ds(jj * 16, 16)
                    ps = src_v[ci, sl]
                    dl = dst_v[ci, sl]
                    pd = dl + coff
                    a = (plsc.load_gather(as_v, [ps])
                         + plsc.load_gather(ad_v, [pd]))
                    a = jnp.where(a > 0.0, a, 0.2 * a)
                    ev = jnp.exp(a - m)
                    ex_v[ci, sl] = ev
                    plsc.addupdate_scatter(
                        den_v, [jnp.right_shift(dl, 4),
                                jnp.bitwise_and(dl, 15)], ev)
                return 0
            lax.fori_loop(0, nca, p1, 0)

            for q in range(DGR // ECH):
                pltpu.sync_copy(den_v.at[pl.ds(q * ECH, ECH)],
                                den_sh.at[iota_v.at[q]], add=True)
            plsc.subcore_barrier()
            pltpu.sync_copy(den_sh, den_v)

            def gstart(ci, buf, sem):
                pltpu.async_copy(h_hbm.at[head].at[src_v.at[ci]], buf, sem)

            def gwait(buf, sem):
                pltpu.make_async_copy(h_hbm.at[head].at[src_v.at[0]],
                                      buf, sem).wait()

            def chunk_compute(ci, buf, sem):
                for jj in range(ECH // 16):
                    sl = pl.ds(jj * 16, 16)
                    dl = dst_v[ci, sl]
                    dg = plsc.load_gather(
                        den_v, [jnp.right_shift(dl, 4),
                                jnp.bitwise_and(dl, 15)])
                    coef_v[sl] = ex_v[ci, sl] / (dg + 1e-16)
                gwait(buf, sem)
                for jj in range(ECH // 16):
                    cvec = coef_v[pl.ds(jj * 16, 16)]
                    for l in range(16):
                        j = jj * 16 + l
                        cf = cvec[l]
                        for w in range(cpad // 16):
                            sl2 = pl.ds(w * 16, 16)
                            buf[j, sl2] = buf[j, sl2] * cf
                pltpu.sync_copy(buf, acc_sh.at[dst_v.at[ci]], add=True)

            @pl.when(nca > 0)
            def _():
                gstart(0, rows_v, sem_a)

            def p2body(ci, _):
                @pl.when(jnp.bitwise_and(ci, 1) == 0)
                def _():
                    @pl.when(ci + 1 < nca)
                    def _():
                        gstart(ci + 1, rows_w, sem_b)
                    chunk_compute(ci, rows_v, sem_a)

                @pl.when(jnp.bitwise_and(ci, 1) == 1)
                def _():
                    @pl.when(ci + 1 < nca)
                    def _():
                        gstart(ci + 1, rows_v, sem_a)
                    chunk_compute(ci, rows_w, sem_b)
                return 0
            lax.fori_loop(0, nca, p2body, 0)
            plsc.subcore_barrier()

            pltpu.sync_copy(acc_sh.at[pl.ds(base, ROWS_PT)],
                            out_hbm.at[head, c, pl.ds(base, ROWS_PT)])
            plsc.subcore_barrier()
            return 0

        lax.fori_loop(0, heads, head_body, 0)

    return k(h, asn, adn, srcr, dstr)


def kernel(x, edge_index, W1, a1_src, a1_dst, b1, W2, a2_src, a2_dst, b2,
           W3, a3_src, a3_dst, b3):
    loop = jnp.arange(N, dtype=edge_index.dtype)
    sent = jnp.full((EPAD - ETOT,), N, dtype=edge_index.dtype)
    srcr = jnp.concatenate([edge_index[0], loop, sent]).reshape(16, NCHUNK, ECH)
    dstr = jnp.concatenate([edge_index[1], loop, sent]).reshape(16, NCHUNK, ECH)

    z = jnp.zeros((HALF - 5000, F_IN), jnp.float32)
    xp = jnp.concatenate([x[0::2], z, x[1::2], z])

    h1, as1, ad1 = _dense_layer1(xp, W1, a1_src, a1_dst)
    acc1 = _sc_edge_layer(h1, as1, ad1, srcr, dstr, 7, HID)
    acc1 = acc1.reshape(7, NP, HID)

    h2, as2, ad2 = _dense_later(acc1, b1, W2, a2_src, a2_dst,
                                hp=7, cp=HID, heads=6, cout=HID, cpad=HID)
    acc2 = _sc_edge_layer(h2, as2, ad2, srcr, dstr, 6, HID).reshape(6, NP, HID)

    h3, as3, ad3 = _dense_later(acc2, b2, W3, a3_src, a3_dst,
                                hp=6, cp=HID, heads=6, cout=NCLS, cpad=48)
    acc3 = _sc_edge_layer(h3, as3, ad3, srcr, dstr, 6, 48)

    out = acc3[:, :, :5000, :NCLS]
    out = out.transpose(2, 1, 0, 3)
    return out.reshape(N, 6 * NCLS) + b3


# speedup vs baseline: 108.7232x; 1.0565x over previous
import functools

import jax
import jax.numpy as jnp
from jax import lax
from jax.experimental import pallas as pl
from jax.experimental.pallas import tpu as pltpu
from jax.experimental.pallas import tpu_sc as plsc

N = 10000
NP = 10240
HALF = 5120
E = 320000
F_IN = 128
HID = 32
NCLS = 40

_BLK = 1024
_GRID = NP // _BLK

ETOT = E + N
ECH = 128
NCHUNK = 162
EPT = NCHUNK * ECH
EPAD = 16 * EPT
ROWS_PT = HALF // 16
DGR = 384
JUNK = 5016


def _dense_body(x, w_ref, asrc_ref, adst_ref, h_ref, as_ref, ad_ref,
                *, heads, cout):
    hfull = jnp.dot(x, w_ref[...], preferred_element_type=jnp.float32)
    asv = []
    adv = []
    for h in range(heads):
        hh = hfull[:, h * cout:(h + 1) * cout]
        if cout == h_ref.shape[2]:
            h_ref[h, :, :] = hh
        else:
            h_ref[h, :, :cout] = hh
            h_ref[h, :, cout:] = jnp.zeros((hh.shape[0], h_ref.shape[2] - cout),
                                           jnp.float32)
        asv.append(jnp.sum(hh * asrc_ref[h][None, :], axis=1))
        adv.append(jnp.sum(hh * adst_ref[h][None, :], axis=1))
    zero = jnp.zeros_like(asv[0])
    while len(asv) < 8:
        asv.append(zero)
        adv.append(zero)
    as_ref[...] = jnp.stack(asv)
    ad_ref[...] = jnp.stack(adv)


def _dense_layer1(x, w, asrc, adst):
    def kfn(x_ref, w_ref, asrc_ref, adst_ref, h_ref, as_ref, ad_ref):
        _dense_body(x_ref[...], w_ref, asrc_ref, adst_ref, h_ref, as_ref,
                    ad_ref, heads=7, cout=HID)

    return pl.pallas_call(
        kfn,
        grid=(_GRID,),
        in_specs=[
            pl.BlockSpec((_BLK, F_IN), lambda i: (i, 0)),
            pl.BlockSpec((F_IN, 7 * HID), lambda i: (0, 0)),
            pl.BlockSpec((7, HID), lambda i: (0, 0)),
            pl.BlockSpec((7, HID), lambda i: (0, 0)),
        ],
        out_specs=[
            pl.BlockSpec((7, _BLK, HID), lambda i: (0, i, 0)),
            pl.BlockSpec((8, _BLK), lambda i: (0, i)),
            pl.BlockSpec((8, _BLK), lambda i: (0, i)),
        ],
        out_shape=[
            jax.ShapeDtypeStruct((7, NP, HID), jnp.float32),
            jax.ShapeDtypeStruct((8, NP), jnp.float32),
            jax.ShapeDtypeStruct((8, NP), jnp.float32),
        ],
    )(x, w, asrc, adst)


def _dense_later(acc_prev, b_prev, w, asrc, adst, *, hp, cp, heads, cout, cpad):
    def kfn(acc_ref, b_ref, w_ref, asrc_ref, adst_ref, h_ref, as_ref, ad_ref):
        parts = [jnp.maximum(acc_ref[h] + b_ref[pl.ds(h * cp, cp)][None, :], 0.0)
                 for h in range(hp)]
        x = jnp.concatenate(parts, axis=1)
        _dense_body(x, w_ref, asrc_ref, adst_ref, h_ref, as_ref, ad_ref,
                    heads=heads, cout=cout)

    return pl.pallas_call(
        kfn,
        grid=(_GRID,),
        in_specs=[
            pl.BlockSpec((hp, _BLK, cp), lambda i: (0, i, 0)),
            pl.BlockSpec((hp * cp,), lambda i: (0,)),
            pl.BlockSpec((hp * cp, heads * cout), lambda i: (0, 0)),
            pl.BlockSpec((heads, cout), lambda i: (0, 0)),
            pl.BlockSpec((heads, cout), lambda i: (0, 0)),
        ],
        out_specs=[
            pl.BlockSpec((heads, _BLK, cpad), lambda i: (0, i, 0)),
            pl.BlockSpec((8, _BLK), lambda i: (0, i)),
            pl.BlockSpec((8, _BLK), lambda i: (0, i)),
        ],
        out_shape=[
            jax.ShapeDtypeStruct((heads, NP, cpad), jnp.float32),
            jax.ShapeDtypeStruct((8, NP), jnp.float32),
            jax.ShapeDtypeStruct((8, NP), jnp.float32),
        ],
    )(acc_prev, b_prev, w, asrc, adst)


def _sc_edge_layer(h, asn, adn, srcr, dstr, heads, cpad):
    mesh = plsc.VectorSubcoreMesh(core_axis_name="c", subcore_axis_name="s")

    @functools.partial(
        pl.kernel,
        out_type=jax.ShapeDtypeStruct((heads, 2, HALF, cpad), jnp.float32),
        mesh=mesh,
        compiler_params=pltpu.CompilerParams(needs_layout_passes=False,
                                             use_tc_tiling_on_sc=False),
        scratch_types=[
            pltpu.VMEM((NCHUNK, ECH), jnp.int32),
            pltpu.VMEM((NCHUNK, ECH), jnp.int32),
            pltpu.VMEM((NCHUNK, ECH), jnp.float32),
            pltpu.VMEM((NP,), jnp.float32),
            pltpu.VMEM((NP,), jnp.float32),
            pltpu.VMEM((DGR, 16), jnp.float32),
            pltpu.VMEM((ECH, cpad), jnp.float32),
            pltpu.VMEM((ECH, cpad), jnp.float32),
            pltpu.VMEM((ECH, cpad), jnp.float32),
            pltpu.VMEM((ECH, cpad), jnp.float32),
            pltpu.VMEM((ECH,), jnp.float32),
            pltpu.SemaphoreType.DMA,
            pltpu.SemaphoreType.DMA,
            pltpu.SemaphoreType.DMA,
            pltpu.SemaphoreType.DMA,
            pltpu.SemaphoreType.DMA,
            pltpu.SemaphoreType.DMA,
            pltpu.VMEM((DGR // ECH, ECH), jnp.int32),
            pltpu.VMEM((DGR // 16, 16), jnp.float32),
            pltpu.VMEM_SHARED((HALF, cpad), jnp.float32),
            pltpu.VMEM_SHARED((DGR, 16), jnp.float32),
        ],
    )
    def k(h_hbm, as_hbm, ad_hbm, src_hbm, dst_hbm, out_hbm,
          src_v, dst_v, ex_v, as_v, ad_v, den_v, rows_v, rows_w, rows_u,
          zrow_v, coef_v, sem_a, sem_b, sem_c, sem_sa, sem_sb, sem_sc,
          iota_v, zq_v, acc_sh, den_sh):
        c = lax.axis_index("c")
        s = lax.axis_index("s")
        zv = jnp.zeros((16,), jnp.float32)
        pltpu.sync_copy(src_hbm.at[s], src_v)
        pltpu.sync_copy(dst_hbm.at[s], dst_v)

        iota16 = lax.iota(jnp.int32, 16)

        def perm_body(v, wp):
            ci = jnp.right_shift(v, 3)
            jo = jnp.bitwise_and(v, 7) * 16
            sv = src_v[ci, pl.ds(jo, 16)]
            dv = dst_v[ci, pl.ds(jo, 16)]
            keep = jnp.bitwise_and(dv, 1) == c
            ps = jnp.bitwise_and(sv, 1) * HALF + jnp.right_shift(sv, 1)
            dl = jnp.right_shift(dv, 1)
            pos = wp - 1 + plsc.cumsum(keep.astype(jnp.int32))
            plsc.store_scatter(src_v, [jnp.right_shift(pos, 7),
                                       jnp.bitwise_and(pos, 127)], ps,
                               mask=keep)
            plsc.store_scatter(dst_v, [jnp.right_shift(pos, 7),
                                       jnp.bitwise_and(pos, 127)], dl,
                               mask=keep)
            cnt = plsc.all_reduce_population_count(keep)
            return wp + cnt[0]
        la = lax.fori_loop(0, NCHUNK * (ECH // 16), perm_body,
                           jnp.int32(0))
        nca = jnp.right_shift(la + 127, 7)
        for kpad in range(8):
            pidx = la + kpad * 16 + iota16
            pmsk = pidx < nca * ECH
            plsc.store_scatter(src_v, [jnp.right_shift(pidx, 7),
                                       jnp.bitwise_and(pidx, 127)],
                               jnp.full((16,), 5000, jnp.int32) + c * HALF,
                               mask=pmsk)
            plsc.store_scatter(dst_v, [jnp.right_shift(pidx, 7),
                                       jnp.bitwise_and(pidx, 127)],
                               jnp.full((16,), JUNK, jnp.int32),
                               mask=pmsk)

        def zrow_body(j, _):
            for w in range(cpad // 16):
                zrow_v[j, pl.ds(w * 16, 16)] = zv
            return 0
        lax.fori_loop(0, ECH, zrow_body, 0)

        def zq_body(j, _):
            zq_v[j, :] = zv
            return 0
        lax.fori_loop(0, DGR // 16, zq_body, 0)

        for q in range(DGR // ECH):
            for jj in range(ECH // 16):
                iota_v[q, pl.ds(jj * 16, 16)] = (
                    lax.iota(jnp.int32, 16) + (q * ECH + jj * 16))

        def head_body(head, _):
            base = s * ROWS_PT
            pltpu.sync_copy(as_hbm.at[head], as_v)
            pltpu.sync_copy(ad_hbm.at[head], ad_v)
            lane8 = lax.iota(jnp.int32, 16) < 8
            for zb in (4992, 10112):
                for ref in (as_v, ad_v):
                    v = ref[pl.ds(zb, 16)]
                    ref[pl.ds(zb, 16)] = jnp.where(lane8, v, 0.0)
                    for t in range(zb + 16, zb + 128, 16):
                        ref[pl.ds(t, 16)] = zv
            pltpu.sync_copy(zrow_v, acc_sh.at[pl.ds(base, ECH)])
            pltpu.sync_copy(zrow_v, acc_sh.at[pl.ds(base + ECH, ECH)])
            pltpu.sync_copy(zrow_v.at[pl.ds(0, 64)],
                            acc_sh.at[pl.ds(base + 2 * ECH, 64)])
            pltpu.sync_copy(zq_v,
                            den_sh.at[pl.ds(s * (DGR // 16), DGR // 16)])

            def zden(t, _):
                den_v[t, :] = zv
                return 0
            lax.fori_loop(0, DGR, zden, 0)

            def mbody(t, carry):
                ma, md = carry
                ma = jnp.maximum(ma, as_v[pl.ds(t * 16, 16)])
                md = jnp.maximum(md, ad_v[pl.ds(t * 16, 16)])
                return ma, md
            minit = jnp.full((16,), -1e30, jnp.float32)
            ma, md = lax.fori_loop(0, NP // 16, mbody, (minit, minit))
            msa = ma[0]
            msd = md[0]
            for l in range(1, 16):
                msa = jnp.maximum(msa, ma[l])
                msd = jnp.maximum(msd, md[l])
            mm = msa + msd
            m = jnp.where(mm > 0.0, mm, 0.2 * mm)

            plsc.subcore_barrier()

            coff = c * HALF

            def p1(ci, _):
                for jj in range(ECH // 16):
                    sl = pl.ds(jj * 16, 16)
                    ps = src_v[ci, sl]
                    dl = dst_v[ci, sl]
                    pd = dl + coff
                    a = (plsc.load_gather(as_v, [ps])
                         + plsc.load_gather(ad_v, [pd]))
                    a = jnp.where(a > 0.0, a, 0.2 * a)
                    ev = jnp.exp(a - m)
                    ex_v[ci, sl] = ev
                    plsc.addupdate_scatter(
                        den_v, [jnp.right_shift(dl, 4),
                                jnp.bitwise_and(dl, 15)], ev)
                return 0
            lax.fori_loop(0, nca, p1, 0)

            for q in range(DGR // ECH):
                pltpu.sync_copy(den_v.at[pl.ds(q * ECH, ECH)],
                                den_sh.at[iota_v.at[q]], add=True)
            plsc.subcore_barrier()
            pltpu.sync_copy(den_sh, den_v)

            bufs = (rows_v, rows_w, rows_u)
            gsems = (sem_a, sem_b, sem_c)
            ssems = (sem_sa, sem_sb, sem_sc)

            def gstart(ci, buf, sem):
                pltpu.async_copy(h_hbm.at[head].at[src_v.at[ci]], buf, sem)

            def gwait(buf, sem):
                pltpu.make_async_copy(h_hbm.at[head].at[src_v.at[0]],
                                      buf, sem).wait()

            def swait(buf, sem):
                pltpu.make_async_copy(buf, acc_sh.at[pl.ds(0, ECH)],
                                      sem).wait()

            def chunk_compute(ci, buf, gsem, ssem):
                for jj in range(ECH // 16):
                    sl = pl.ds(jj * 16, 16)
                    dl = dst_v[ci, sl]
                    dg = plsc.load_gather(
                        den_v, [jnp.right_shift(dl, 4),
                                jnp.bitwise_and(dl, 15)])
                    coef_v[sl] = ex_v[ci, sl] / (dg + 1e-16)
                gwait(buf, gsem)
                for jj in range(ECH // 16):
                    cvec = coef_v[pl.ds(jj * 16, 16)]
                    for l in range(16):
                        j = jj * 16 + l
                        cf = cvec[l]
                        for w in range(cpad // 16):
                            sl2 = pl.ds(w * 16, 16)
                            buf[j, sl2] = buf[j, sl2] * cf
                pltpu.async_copy(buf, acc_sh.at[dst_v.at[ci]], ssem,
                                 add=True)

            @pl.when(nca > 0)
            def _():
                gstart(0, rows_v, sem_a)

            def p2body(ci, _):
                rem = lax.rem(ci, 3)
                for kb in range(3):
                    nb = (kb + 1) % 3

                    @pl.when(rem == kb)
                    def _():
                        @pl.when(ci + 1 < nca)
                        def _():
                            @pl.when(ci >= 2)
                            def _():
                                swait(bufs[nb], ssems[nb])
                            gstart(ci + 1, bufs[nb], gsems[nb])
                        chunk_compute(ci, bufs[kb], gsems[kb], ssems[kb])
                return 0
            lax.fori_loop(0, nca, p2body, 0)
            for kb in range(3):
                @pl.when(nca > kb)
                def _():
                    swait(bufs[kb], ssems[kb])
            plsc.subcore_barrier()

            pltpu.sync_copy(acc_sh.at[pl.ds(base, ROWS_PT)],
                            out_hbm.at[head, c, pl.ds(base, ROWS_PT)])
            plsc.subcore_barrier()
            return 0

        lax.fori_loop(0, heads, head_body, 0)

    return k(h, asn, adn, srcr, dstr)


def kernel(x, edge_index, W1, a1_src, a1_dst, b1, W2, a2_src, a2_dst, b2,
           W3, a3_src, a3_dst, b3):
    loop = jnp.arange(N, dtype=edge_index.dtype)
    sent = jnp.full((EPAD - ETOT,), N, dtype=edge_index.dtype)
    srcr = jnp.concatenate([edge_index[0], loop, sent]).reshape(16, NCHUNK, ECH)
    dstr = jnp.concatenate([edge_index[1], loop, sent]).reshape(16, NCHUNK, ECH)

    z = jnp.zeros((HALF - 5000, F_IN), jnp.float32)
    xp = jnp.concatenate([x[0::2], z, x[1::2], z])

    h1, as1, ad1 = _dense_layer1(xp, W1, a1_src, a1_dst)
    acc1 = _sc_edge_layer(h1, as1, ad1, srcr, dstr, 7, HID)
    acc1 = acc1.reshape(7, NP, HID)

    h2, as2, ad2 = _dense_later(acc1, b1, W2, a2_src, a2_dst,
                                hp=7, cp=HID, heads=6, cout=HID, cpad=HID)
    acc2 = _sc_edge_layer(h2, as2, ad2, srcr, dstr, 6, HID).reshape(6, NP, HID)

    h3, as3, ad3 = _dense_later(acc2, b2, W3, a3_src, a3_dst,
                                hp=6, cp=HID, heads=6, cout=NCLS, cpad=48)
    acc3 = _sc_edge_layer(h3, as3, ad3, srcr, dstr, 6, 48)

    out = acc3[:, :, :5000, :NCLS]
    out = out.transpose(2, 1, 0, 3)
    return out.reshape(N, 6 * NCLS) + b3
